# baseline (speedup 1.0000x reference)
import jax
import jax.numpy as jnp
from jax import lax
from jax.experimental import pallas as pl
from jax.experimental.pallas import tpu as pltpu
from jax.experimental.pallas import tpu_sc as plsc

N = 10000
E = 320000
D = 128
H = 64
G = 128

NTILES = 32
ROWS_PER_TILE = 640
NPAD = NTILES * ROWS_PER_TILE // 2
CHUNK = 128
NCHUNK = 80
ZROWS = 64
EPAD = NTILES * NCHUNK * CHUNK

_f32 = jnp.float32


def _make_sc_body(w, nbuf):
    def body(t_hbm, src_hbm, dst_hbm, out_hbm, idx_s, idx_d, rows, zbuf,
             acc, sem):
        c = lax.axis_index("c")
        s = lax.axis_index("s")
        wid = c * 16 + s

        def _zrow(i, carry):
            for k in range(w // 16):
                zbuf[i, pl.ds(k * 16, 16)] = jnp.zeros((16,), _f32)
            return carry

        lax.fori_loop(0, ZROWS, _zrow, 0)

        def _zcopy(i, carry):
            pltpu.sync_copy(
                zbuf, acc.at[pl.ds(s * ROWS_PER_TILE + i * ZROWS, ZROWS)])
            return carry

        lax.fori_loop(0, ROWS_PER_TILE // ZROWS, _zcopy, 0)

        pltpu.sync_copy(src_hbm.at[wid], idx_s)
        pltpu.sync_copy(dst_hbm.at[wid], idx_d)
        plsc.subcore_barrier()

        for b in range(nbuf):
            pltpu.async_copy(t_hbm.at[idx_s.at[b]], rows.at[b], sem.at[b])

        def _grp(g, carry):
            j0 = g * nbuf
            for b in range(nbuf):
                j = j0 + b
                pltpu.make_async_copy(t_hbm.at[idx_s.at[j]], rows.at[b],
                                      sem.at[b]).wait()
                pltpu.sync_copy(rows.at[b], acc.at[idx_d.at[j]], add=True)

                @pl.when(j + nbuf < NCHUNK)
                def _():
                    pltpu.async_copy(t_hbm.at[idx_s.at[j + nbuf]],
                                     rows.at[b], sem.at[b])
            return carry

        lax.fori_loop(0, NCHUNK // nbuf, _grp, 0)
        plsc.subcore_barrier()

        sl = pl.ds(s * ROWS_PER_TILE, ROWS_PER_TILE)
        pltpu.sync_copy(acc.at[sl], out_hbm.at[c].at[sl])

    return body


def _make_sc_segsum(w, nbuf):
    body = _make_sc_body(w, nbuf)

    def segsum(t, src_r, dst_r):
        mesh = plsc.VectorSubcoreMesh(core_axis_name="c",
                                      subcore_axis_name="s")
        kfn = pl.kernel(
            body,
            out_type=jax.ShapeDtypeStruct((2, NPAD, w), _f32),
            mesh=mesh,
            scratch_types=[
                pltpu.VMEM((NCHUNK, CHUNK), jnp.int32),
                pltpu.VMEM((NCHUNK, CHUNK), jnp.int32),
                pltpu.VMEM((nbuf, CHUNK, w), _f32),
                pltpu.VMEM((ZROWS, w), _f32),
                pltpu.VMEM_SHARED((NPAD, w), _f32),
                pltpu.SemaphoreType.DMA((nbuf,)),
            ],
            compiler_params=pltpu.CompilerParams(use_tc_tiling_on_sc=False),
        )
        return kfn(t, src_r, dst_r)

    return segsum


_sc_segsum_narrow = _make_sc_segsum(H, 4)


def _bf(x):
    return x.astype(jnp.bfloat16)


def _group_mask(batch_row):
    g_iota = lax.broadcasted_iota(jnp.int32, (G, NPAD), 0)
    return (batch_row[None, :] == g_iota).astype(_f32)


def _masked_bn(z, nmask, g, b):
    mu = jnp.sum(z * nmask, axis=0, keepdims=True) * (1.0 / N)
    zc = z - mu
    var = jnp.sum(zc * zc * nmask, axis=0, keepdims=True) * (1.0 / N)
    v = var + 1e-5
    r0 = lax.rsqrt(v)
    r = r0 * (1.5 - 0.5 * v * r0 * r0)
    return g * zc * r + b


def _tc_pool0_body(x_ref, batch_ref, pool_ref):
    pool_ref[...] = jnp.dot(_group_mask(batch_ref[...]), x_ref[...],
                            preferred_element_type=_f32,
                            precision=lax.Precision.HIGHEST)


def _tc_conv_body(t_ref, parts_ref, zacc_ref, xr_ref, beta, is0,
                  w1, b1, g1, be1, w2, b2, g2, be2,
                  batch_ref, tn_ref, zpart_ref, pool_ref):
    nmask = (lax.broadcasted_iota(jnp.int32, (NPAD, 1), 0) < N).astype(_f32)
    t = t_ref[...] + parts_ref[0] + parts_ref[1]
    zpart = jnp.dot(_bf(t), _bf(w1[...]), preferred_element_type=_f32)
    zpart_ref[...] = zpart
    z = beta[...] * zacc_ref[...] + zpart + b1[...]
    z = _masked_bn(z, nmask, g1[...], be1[...])
    r = jnp.maximum(z, 0.0) * nmask
    v = jnp.dot(_bf(r), _bf(w2[...]), preferred_element_type=_f32) + b2[...]
    v = _masked_bn(v, nmask, g2[...], be2[...])
    h = jnp.maximum(v, 0.0) * nmask
    tn_ref[...] = jnp.where(is0[...] > 0, xr_ref[...], h)
    pool_ref[...] = jnp.dot(_group_mask(batch_ref[...]), h,
                            preferred_element_type=_f32,
                            precision=lax.Precision.HIGHEST)


def _tc_head_body(p0, p1, p2, p3, lw0, lb0, lw1, lb1, y_ref):
    hcat = jnp.concatenate([p0[...], p1[...], p2[...], p3[...]], axis=1)
    y = jnp.dot(_bf(hcat), _bf(lw0[...]), preferred_element_type=_f32) \
        + lb0[...]
    y = jnp.maximum(y, 0.0)
    y_ref[...] = jnp.dot(_bf(y), _bf(lw1[...]), preferred_element_type=_f32) \
        + lb1[...]


def _tc_conv(t, parts, zacc, xr, cp, batch_pad):
    return pl.pallas_call(
        _tc_conv_body,
        out_shape=[jax.ShapeDtypeStruct((NPAD, H), _f32),
                   jax.ShapeDtypeStruct((NPAD, H), _f32),
                   jax.ShapeDtypeStruct((G, H), _f32)],
    )(t, parts, zacc, xr, cp["beta"], cp["is0"], cp["w1"], cp["b1"],
      cp["bng"], cp["bnb"], cp["w2"], cp["b2"], cp["og"], cp["ob"],
      batch_pad)


def kernel(x, edge_index, batch, params):
    x_pad = jnp.zeros((NPAD, D), _f32).at[:N].set(x)
    batch_pad = jnp.zeros((NPAD,), jnp.int32).at[:N].set(batch)
    pad_pt = NCHUNK * CHUNK - E // NTILES
    fake = jnp.broadcast_to(
        (N + jnp.arange(pad_pt, dtype=jnp.int32))[None], (NTILES, pad_pt))
    src_r = jnp.concatenate(
        [edge_index[0].reshape(NTILES, -1), fake], axis=1).reshape(
            NTILES, NCHUNK, CHUNK)
    dst_r = jnp.concatenate(
        [edge_index[1].reshape(NTILES, -1), fake], axis=1).reshape(
            NTILES, NCHUNK, CHUNK)

    row = lambda v: v.reshape(1, -1)

    pool0 = pl.pallas_call(
        _tc_pool0_body,
        out_shape=jax.ShapeDtypeStruct((G, D), _f32),
    )(x_pad, batch_pad)

    convs = params["convs"]
    obn = params["obn"]
    zero = jnp.zeros((1, H), _f32)
    one = jnp.ones((1, H), _f32)
    w1_0 = convs[0]["w1"]

    def stk(*xs):
        return jnp.stack(xs)

    cps = {
        "beta": stk(*(jnp.full((1, 1), v, _f32) for v in (0.0, 1.0, 0.0, 0.0))),
        "is0": stk(*(jnp.full((1, 1), v, _f32) for v in (1.0, 0.0, 0.0, 0.0))),
        "w1": stk(w1_0[:H], w1_0[H:], convs[1]["w1"], convs[2]["w1"]),
        "b1": stk(zero, row(convs[0]["b1"]), row(convs[1]["b1"]),
                  row(convs[2]["b1"])),
        "bng": stk(one, row(convs[0]["bng"]), row(convs[1]["bng"]),
                   row(convs[2]["bng"])),
        "bnb": stk(zero, row(convs[0]["bnb"]), row(convs[1]["bnb"]),
                   row(convs[2]["bnb"])),
        "w2": stk(convs[0]["w2"], convs[0]["w2"], convs[1]["w2"],
                  convs[2]["w2"]),
        "b2": stk(zero, row(convs[0]["b2"]), row(convs[1]["b2"]),
                  row(convs[2]["b2"])),
        "og": stk(one, row(obn[0]["g"]), row(obn[1]["g"]), row(obn[2]["g"])),
        "ob": stk(zero, row(obn[0]["b"]), row(obn[1]["b"]), row(obn[2]["b"])),
    }

    x_l = x_pad[:, :H]
    x_r = x_pad[:, H:]

    def _step(carry, cp):
        t, zacc = carry
        parts = _sc_segsum_narrow(t, src_r, dst_r)
        tn, zpart, pool = _tc_conv(t, parts, zacc, x_r, cp, batch_pad)
        return (tn, zpart), pool

    _, pools = lax.scan(_step, (x_l, jnp.zeros((NPAD, H), _f32)), cps)

    lins = params["lins"]
    y = pl.pallas_call(
        _tc_head_body,
        out_shape=jax.ShapeDtypeStruct((G, 1), _f32),
    )(pool0, pools[1], pools[2], pools[3],
      lins[0]["w"], row(lins[0]["b"]), lins[1]["w"], row(lins[1]["b"]))
    return y.reshape(-1)


# speedup vs baseline: 1.0019x; 1.0019x over previous
import jax
import jax.numpy as jnp
from jax import lax
from jax.experimental import pallas as pl
from jax.experimental.pallas import tpu as pltpu
from jax.experimental.pallas import tpu_sc as plsc

N = 10000
E = 320000
D = 128
H = 64
G = 128

NTILES = 32
ROWS_PER_TILE = 640
NPAD = NTILES * ROWS_PER_TILE // 2
CHUNK = 128
NCHUNK = 80
ZROWS = 64
EPAD = NTILES * NCHUNK * CHUNK

_f32 = jnp.float32


def _make_sc_body(w, nbuf):
    def body(t_hbm, src_hbm, dst_hbm, out_hbm, idx_s, idx_d, rows, zbuf,
             acc, sem):
        c = lax.axis_index("c")
        s = lax.axis_index("s")
        wid = c * 16 + s

        def _zrow(i, carry):
            for k in range(w // 16):
                zbuf[i, pl.ds(k * 16, 16)] = jnp.zeros((16,), _f32)
            return carry

        lax.fori_loop(0, ZROWS, _zrow, 0)

        def _zcopy(i, carry):
            pltpu.sync_copy(
                zbuf, acc.at[pl.ds(s * ROWS_PER_TILE + i * ZROWS, ZROWS)])
            return carry

        lax.fori_loop(0, ROWS_PER_TILE // ZROWS, _zcopy, 0)

        pltpu.sync_copy(src_hbm.at[wid], idx_s)
        pltpu.sync_copy(dst_hbm.at[wid], idx_d)
        plsc.subcore_barrier()

        for b in range(nbuf):
            pltpu.async_copy(t_hbm.at[idx_s.at[b]], rows.at[b], sem.at[b])

        def _grp(g, carry):
            j0 = g * nbuf
            for b in range(nbuf):
                j = j0 + b
                pltpu.make_async_copy(t_hbm.at[idx_s.at[j]], rows.at[b],
                                      sem.at[b]).wait()
                pltpu.sync_copy(rows.at[b], acc.at[idx_d.at[j]], add=True)

                @pl.when(j + nbuf < NCHUNK)
                def _():
                    pltpu.async_copy(t_hbm.at[idx_s.at[j + nbuf]],
                                     rows.at[b], sem.at[b])
            return carry

        lax.fori_loop(0, NCHUNK // nbuf, _grp, 0)
        plsc.subcore_barrier()

        sl = pl.ds(s * ROWS_PER_TILE, ROWS_PER_TILE)
        pltpu.sync_copy(acc.at[sl], out_hbm.at[c].at[sl])

    return body


def _make_sc_segsum(w, nbuf):
    body = _make_sc_body(w, nbuf)

    def segsum(t, src_r, dst_r):
        mesh = plsc.VectorSubcoreMesh(core_axis_name="c",
                                      subcore_axis_name="s")
        kfn = pl.kernel(
            body,
            out_type=jax.ShapeDtypeStruct((2, NPAD, w), _f32),
            mesh=mesh,
            scratch_types=[
                pltpu.VMEM((NCHUNK, CHUNK), jnp.int32),
                pltpu.VMEM((NCHUNK, CHUNK), jnp.int32),
                pltpu.VMEM((nbuf, CHUNK, w), _f32),
                pltpu.VMEM((ZROWS, w), _f32),
                pltpu.VMEM_SHARED((NPAD, w), _f32),
                pltpu.SemaphoreType.DMA((nbuf,)),
            ],
            compiler_params=pltpu.CompilerParams(use_tc_tiling_on_sc=False),
        )
        return kfn(t, src_r, dst_r)

    return segsum


_sc_segsum_narrow = _make_sc_segsum(H, 8)


def _bf(x):
    return x.astype(jnp.bfloat16)


def _group_mask(batch_row):
    g_iota = lax.broadcasted_iota(jnp.int32, (G, NPAD), 0)
    return (batch_row[None, :] == g_iota).astype(_f32)


def _masked_bn(z, nmask, g, b):
    mu = jnp.sum(z * nmask, axis=0, keepdims=True) * (1.0 / N)
    zc = z - mu
    var = jnp.sum(zc * zc * nmask, axis=0, keepdims=True) * (1.0 / N)
    v = var + 1e-5
    r0 = lax.rsqrt(v)
    r = r0 * (1.5 - 0.5 * v * r0 * r0)
    return g * zc * r + b


def _tc_pool0_body(x_ref, batch_ref, pool_ref):
    pool_ref[...] = jnp.dot(_group_mask(batch_ref[...]), x_ref[...],
                            preferred_element_type=_f32,
                            precision=lax.Precision.HIGHEST)


def _tc_conv_body(t_ref, parts_ref, zacc_ref, xr_ref, beta, is0,
                  w1, b1, g1, be1, w2, b2, g2, be2,
                  batch_ref, tn_ref, zpart_ref, pool_ref):
    nmask = (lax.broadcasted_iota(jnp.int32, (NPAD, 1), 0) < N).astype(_f32)
    t = t_ref[...] + parts_ref[0] + parts_ref[1]
    zpart = jnp.dot(_bf(t), _bf(w1[...]), preferred_element_type=_f32)
    zpart_ref[...] = zpart
    z = beta[...] * zacc_ref[...] + zpart + b1[...]
    z = _masked_bn(z, nmask, g1[...], be1[...])
    r = jnp.maximum(z, 0.0) * nmask
    v = jnp.dot(_bf(r), _bf(w2[...]), preferred_element_type=_f32) + b2[...]
    v = _masked_bn(v, nmask, g2[...], be2[...])
    h = jnp.maximum(v, 0.0) * nmask
    tn_ref[...] = jnp.where(is0[...] > 0, xr_ref[...], h)
    pool_ref[...] = jnp.dot(_group_mask(batch_ref[...]), h,
                            preferred_element_type=_f32,
                            precision=lax.Precision.HIGHEST)


def _tc_head_body(p0, p1, p2, p3, lw0, lb0, lw1, lb1, y_ref):
    hcat = jnp.concatenate([p0[...], p1[...], p2[...], p3[...]], axis=1)
    y = jnp.dot(_bf(hcat), _bf(lw0[...]), preferred_element_type=_f32) \
        + lb0[...]
    y = jnp.maximum(y, 0.0)
    y_ref[...] = jnp.dot(_bf(y), _bf(lw1[...]), preferred_element_type=_f32) \
        + lb1[...]


def _tc_conv(t, parts, zacc, xr, cp, batch_pad):
    return pl.pallas_call(
        _tc_conv_body,
        out_shape=[jax.ShapeDtypeStruct((NPAD, H), _f32),
                   jax.ShapeDtypeStruct((NPAD, H), _f32),
                   jax.ShapeDtypeStruct((G, H), _f32)],
    )(t, parts, zacc, xr, cp["beta"], cp["is0"], cp["w1"], cp["b1"],
      cp["bng"], cp["bnb"], cp["w2"], cp["b2"], cp["og"], cp["ob"],
      batch_pad)


def kernel(x, edge_index, batch, params):
    x_pad = jnp.zeros((NPAD, D), _f32).at[:N].set(x)
    batch_pad = jnp.zeros((NPAD,), jnp.int32).at[:N].set(batch)
    pad_pt = NCHUNK * CHUNK - E // NTILES
    fake = jnp.broadcast_to(
        (N + jnp.arange(pad_pt, dtype=jnp.int32))[None], (NTILES, pad_pt))
    src_r = jnp.concatenate(
        [edge_index[0].reshape(NTILES, -1), fake], axis=1).reshape(
            NTILES, NCHUNK, CHUNK)
    dst_r = jnp.concatenate(
        [edge_index[1].reshape(NTILES, -1), fake], axis=1).reshape(
            NTILES, NCHUNK, CHUNK)

    row = lambda v: v.reshape(1, -1)

    pool0 = pl.pallas_call(
        _tc_pool0_body,
        out_shape=jax.ShapeDtypeStruct((G, D), _f32),
    )(x_pad, batch_pad)

    convs = params["convs"]
    obn = params["obn"]
    zero = jnp.zeros((1, H), _f32)
    one = jnp.ones((1, H), _f32)
    w1_0 = convs[0]["w1"]

    def stk(*xs):
        return jnp.stack(xs)

    cps = {
        "beta": stk(*(jnp.full((1, 1), v, _f32) for v in (0.0, 1.0, 0.0, 0.0))),
        "is0": stk(*(jnp.full((1, 1), v, _f32) for v in (1.0, 0.0, 0.0, 0.0))),
        "w1": stk(w1_0[:H], w1_0[H:], convs[1]["w1"], convs[2]["w1"]),
        "b1": stk(zero, row(convs[0]["b1"]), row(convs[1]["b1"]),
                  row(convs[2]["b1"])),
        "bng": stk(one, row(convs[0]["bng"]), row(convs[1]["bng"]),
                   row(convs[2]["bng"])),
        "bnb": stk(zero, row(convs[0]["bnb"]), row(convs[1]["bnb"]),
                   row(convs[2]["bnb"])),
        "w2": stk(convs[0]["w2"], convs[0]["w2"], convs[1]["w2"],
                  convs[2]["w2"]),
        "b2": stk(zero, row(convs[0]["b2"]), row(convs[1]["b2"]),
                  row(convs[2]["b2"])),
        "og": stk(one, row(obn[0]["g"]), row(obn[1]["g"]), row(obn[2]["g"])),
        "ob": stk(zero, row(obn[0]["b"]), row(obn[1]["b"]), row(obn[2]["b"])),
    }

    x_l = x_pad[:, :H]
    x_r = x_pad[:, H:]

    def _step(carry, cp):
        t, zacc = carry
        parts = _sc_segsum_narrow(t, src_r, dst_r)
        tn, zpart, pool = _tc_conv(t, parts, zacc, x_r, cp, batch_pad)
        return (tn, zpart), pool

    _, pools = lax.scan(_step, (x_l, jnp.zeros((NPAD, H), _f32)), cps)

    lins = params["lins"]
    y = pl.pallas_call(
        _tc_head_body,
        out_shape=jax.ShapeDtypeStruct((G, 1), _f32),
    )(pool0, pools[1], pools[2], pools[3],
      lins[0]["w"], row(lins[0]["b"]), lins[1]["w"], row(lins[1]["b"]))
    return y.reshape(-1)


# speedup vs baseline: 1.1759x; 1.1736x over previous
import jax
import jax.numpy as jnp
from jax import lax
from jax.experimental import pallas as pl
from jax.experimental.pallas import tpu as pltpu
from jax.experimental.pallas import tpu_sc as plsc

N = 10000
E = 320000
D = 128
H = 64
G = 128

NTILES = 32
ROWS_PER_TILE = 640
NPAD = NTILES * ROWS_PER_TILE // 2
CHUNK = 128
NCHUNK = 80
ZROWS = 64
EPAD = NTILES * NCHUNK * CHUNK

_f32 = jnp.float32


def _make_sc_body(w, nbuf, npass):
    def body(t_hbm, src_hbm, dst_hbm, out_hbm, idx_s, idx_d, rows, zbuf,
             acc, sem):
        c = lax.axis_index("c")
        s = lax.axis_index("s")
        wid = c * 16 + s

        def _zrow(i, carry):
            for k in range(w // 16):
                zbuf[i, pl.ds(k * 16, 16)] = jnp.zeros((16,), _f32)
            return carry

        lax.fori_loop(0, ZROWS, _zrow, 0)

        pltpu.sync_copy(src_hbm.at[wid], idx_s)
        pltpu.sync_copy(dst_hbm.at[wid], idx_d)

        sl = pl.ds(s * ROWS_PER_TILE, ROWS_PER_TILE)
        for p in range(npass):
            tp = t_hbm.at[p] if npass > 1 else t_hbm

            def _zcopy(i, carry):
                pltpu.sync_copy(
                    zbuf, acc.at[pl.ds(s * ROWS_PER_TILE + i * ZROWS,
                                       ZROWS)])
                return carry

            lax.fori_loop(0, ROWS_PER_TILE // ZROWS, _zcopy, 0)
            plsc.subcore_barrier()

            for b in range(nbuf):
                pltpu.async_copy(tp.at[idx_s.at[b]], rows.at[b], sem.at[b])

            def _grp(g, carry):
                j0 = g * nbuf
                for b in range(nbuf):
                    j = j0 + b
                    pltpu.make_async_copy(tp.at[idx_s.at[j]], rows.at[b],
                                          sem.at[b]).wait()
                    pltpu.sync_copy(rows.at[b], acc.at[idx_d.at[j]],
                                    add=True)

                    @pl.when(j + nbuf < NCHUNK)
                    def _():
                        pltpu.async_copy(tp.at[idx_s.at[j + nbuf]],
                                         rows.at[b], sem.at[b])
                return carry

            lax.fori_loop(0, NCHUNK // nbuf, _grp, 0)
            plsc.subcore_barrier()

            pltpu.sync_copy(acc.at[sl], out_hbm.at[2 * p + c].at[sl])

    return body


def _make_sc_segsum(w, nbuf, npass):
    body = _make_sc_body(w, nbuf, npass)

    def segsum(t, src_r, dst_r):
        mesh = plsc.VectorSubcoreMesh(core_axis_name="c",
                                      subcore_axis_name="s")
        kfn = pl.kernel(
            body,
            out_type=jax.ShapeDtypeStruct((2 * npass, NPAD, w), _f32),
            mesh=mesh,
            scratch_types=[
                pltpu.VMEM((NCHUNK, CHUNK), jnp.int32),
                pltpu.VMEM((NCHUNK, CHUNK), jnp.int32),
                pltpu.VMEM((nbuf, CHUNK, w), _f32),
                pltpu.VMEM((ZROWS, w), _f32),
                pltpu.VMEM_SHARED((NPAD, w), _f32),
                pltpu.SemaphoreType.DMA((nbuf,)),
            ],
            compiler_params=pltpu.CompilerParams(use_tc_tiling_on_sc=False),
        )
        return kfn(t, src_r, dst_r)

    return segsum


_sc_segsum_narrow = _make_sc_segsum(H, 4, 1)
_sc_segsum_dual = _make_sc_segsum(H, 4, 2)


def _bf(x):
    return x.astype(jnp.bfloat16)


def _group_mask(batch_row):
    g_iota = lax.broadcasted_iota(jnp.int32, (G, NPAD), 0)
    return (batch_row[None, :] == g_iota).astype(_f32)


def _masked_bn(z, nmask, g, b):
    mu = jnp.sum(z * nmask, axis=0, keepdims=True) * (1.0 / N)
    zc = z - mu
    var = jnp.sum(zc * zc * nmask, axis=0, keepdims=True) * (1.0 / N)
    v = var + 1e-5
    r0 = lax.rsqrt(v)
    r = r0 * (1.5 - 0.5 * v * r0 * r0)
    return g * zc * r + b


def _tc_pool0_body(x_ref, batch_ref, pool_ref):
    pool_ref[...] = jnp.dot(_group_mask(batch_ref[...]), x_ref[...],
                            preferred_element_type=_f32,
                            precision=lax.Precision.HIGHEST)


def _conv_tail(z, g1, be1, w2, b2, g2, be2, batch_ref, nmask):
    z = _masked_bn(z, nmask, g1[...], be1[...])
    r = jnp.maximum(z, 0.0) * nmask
    v = jnp.dot(_bf(r), _bf(w2[...]), preferred_element_type=_f32) + b2[...]
    v = _masked_bn(v, nmask, g2[...], be2[...])
    h = jnp.maximum(v, 0.0) * nmask
    pool = jnp.dot(_group_mask(batch_ref[...]), h,
                   preferred_element_type=_f32,
                   precision=lax.Precision.HIGHEST)
    return h, pool


def _tc_conv0_body(x_ref, parts_ref, w1, b1, g1, be1, w2, b2, g2, be2,
                   batch_ref, h_ref, pool_ref):
    nmask = (lax.broadcasted_iota(jnp.int32, (NPAD, 1), 0) < N).astype(_f32)
    agg = jnp.concatenate(
        [parts_ref[0] + parts_ref[1], parts_ref[2] + parts_ref[3]], axis=1)
    t = x_ref[...] + agg
    z = jnp.dot(_bf(t), _bf(w1[...]), preferred_element_type=_f32) + b1[...]
    h, pool = _conv_tail(z, g1, be1, w2, b2, g2, be2, batch_ref, nmask)
    h_ref[...] = h
    pool_ref[...] = pool


def _tc_conv_body(t_ref, parts_ref, w1, b1, g1, be1, w2, b2, g2, be2,
                  batch_ref, h_ref, pool_ref):
    nmask = (lax.broadcasted_iota(jnp.int32, (NPAD, 1), 0) < N).astype(_f32)
    t = t_ref[...] + parts_ref[0] + parts_ref[1]
    z = jnp.dot(_bf(t), _bf(w1[...]), preferred_element_type=_f32) + b1[...]
    h, pool = _conv_tail(z, g1, be1, w2, b2, g2, be2, batch_ref, nmask)
    h_ref[...] = h
    pool_ref[...] = pool


def _tc_head_body(p0, p1, p2, p3, lw0, lb0, lw1, lb1, y_ref):
    hcat = jnp.concatenate([p0[...], p1[...], p2[...], p3[...]], axis=1)
    y = jnp.dot(_bf(hcat), _bf(lw0[...]), preferred_element_type=_f32) \
        + lb0[...]
    y = jnp.maximum(y, 0.0)
    y_ref[...] = jnp.dot(_bf(y), _bf(lw1[...]), preferred_element_type=_f32) \
        + lb1[...]


def _tc_conv(body, t, parts, cp, batch_pad):
    return pl.pallas_call(
        body,
        out_shape=[jax.ShapeDtypeStruct((NPAD, H), _f32),
                   jax.ShapeDtypeStruct((G, H), _f32)],
    )(t, parts, cp["w1"], cp["b1"], cp["bng"], cp["bnb"], cp["w2"],
      cp["b2"], cp["og"], cp["ob"], batch_pad)


def kernel(x, edge_index, batch, params):
    x_pad = jnp.zeros((NPAD, D), _f32).at[:N].set(x)
    batch_pad = jnp.zeros((NPAD,), jnp.int32).at[:N].set(batch)
    pad_pt = NCHUNK * CHUNK - E // NTILES
    fake = jnp.broadcast_to(
        (N + jnp.arange(pad_pt, dtype=jnp.int32))[None], (NTILES, pad_pt))
    src_r = jnp.concatenate(
        [edge_index[0].reshape(NTILES, -1), fake], axis=1).reshape(
            NTILES, NCHUNK, CHUNK)
    dst_r = jnp.concatenate(
        [edge_index[1].reshape(NTILES, -1), fake], axis=1).reshape(
            NTILES, NCHUNK, CHUNK)

    row = lambda v: v.reshape(1, -1)

    def conv_params(i):
        cp = params["convs"][i]
        ob = params["obn"][i]
        return {"w1": cp["w1"], "b1": row(cp["b1"]), "bng": row(cp["bng"]),
                "bnb": row(cp["bnb"]), "w2": cp["w2"], "b2": row(cp["b2"]),
                "og": row(ob["g"]), "ob": row(ob["b"])}

    pool0 = pl.pallas_call(
        _tc_pool0_body,
        out_shape=jax.ShapeDtypeStruct((G, D), _f32),
    )(x_pad, batch_pad)

    x2 = jnp.stack([x_pad[:, :H], x_pad[:, H:]])
    parts0 = _sc_segsum_dual(x2, src_r, dst_r)
    h1, pool1 = _tc_conv(_tc_conv0_body, x_pad, parts0, conv_params(0),
                         batch_pad)

    cp12 = jax.tree.map(lambda a, b: jnp.stack([a, b]),
                        conv_params(1), conv_params(2))

    def _step(h, cp):
        parts = _sc_segsum_narrow(h, src_r, dst_r)
        hn, pool = _tc_conv(_tc_conv_body, h, parts, cp, batch_pad)
        return hn, pool

    _, pools = lax.scan(_step, h1, cp12)

    lins = params["lins"]
    y = pl.pallas_call(
        _tc_head_body,
        out_shape=jax.ShapeDtypeStruct((G, 1), _f32),
    )(pool0, pool1, pools[0], pools[1],
      lins[0]["w"], row(lins[0]["b"]), lins[1]["w"], row(lins[1]["b"]))
    return y.reshape(-1)
